# baseline (speedup 1.0000x reference)
import functools

import jax
import jax.numpy as jnp
from jax.experimental import pallas as pl
from jax.experimental.pallas import tpu as pltpu

_KH = 5
_PAD = _KH // 2
_INV_ALPHA = 10.0


def _lbp_body(H, W, P, R, L, cs_ref, kys_ref, kxs_ref,
              x0_ref, x1_ref, x2_ref, x3_ref, out_ref):
    f = pl.program_id(0)

    acc = None
    for p, x_ref in enumerate((x0_ref, x1_ref, x2_ref, x3_ref)):
        idx = f * P + p
        ch = x_ref[:, 0]

        ky = kys_ref[idx]
        kx = kxs_ref[idx]
        s = jnp.pad(ch.astype(jnp.bfloat16), ((0, 0), (0, R - H), (0, L - W)))
        s = pltpu.roll(s, ((R + _PAD) - ky) % R, 1)
        s = pltpu.roll(s, ((L + _PAD) - kx) % L, 2)
        nb = s[:, 0:H, 0:W].astype(jnp.float32)

        val = float(2 ** p) * jax.nn.sigmoid((nb - ch) * _INV_ALPHA)
        acc = val if acc is None else acc + val

    out_ref[:, 0] = acc


def kernel(input, kernels, projection_map):
    N, C, H, W = input.shape
    F, P = projection_map.shape

    cs = projection_map.reshape(-1).astype(jnp.int32)
    kys = kernels[..., 0].reshape(-1).astype(jnp.int32)
    kxs = kernels[..., 1].reshape(-1).astype(jnp.int32)

    rows = H + 8
    cols = W + 32

    body = functools.partial(_lbp_body, H, W, P, rows, cols)

    def _in_spec(p):
        return pl.BlockSpec(
            (N, 1, H, W),
            lambda f, cs_r, kys_r, kxs_r: (0, cs_r[f * P + p], 0, 0),
        )

    grid_spec = pltpu.PrefetchScalarGridSpec(
        num_scalar_prefetch=3,
        grid=(F,),
        in_specs=[_in_spec(p) for p in range(P)],
        out_specs=pl.BlockSpec(
            (N, 1, H, W),
            lambda f, cs_r, kys_r, kxs_r: (0, f, 0, 0),
        ),
    )

    return pl.pallas_call(
        body,
        grid_spec=grid_spec,
        out_shape=jax.ShapeDtypeStruct((N, F, H, W), jnp.float32),
        compiler_params=pltpu.CompilerParams(
            dimension_semantics=("arbitrary",),
        ),
    )(cs, kys, kxs, input, input, input, input)


# speedup vs baseline: 1.1713x; 1.1713x over previous
import functools

import jax
import jax.numpy as jnp
from jax.experimental import pallas as pl
from jax.experimental.pallas import tpu as pltpu

_KH = 5
_PAD = _KH // 2
_INV_ALPHA = 10.0


def _lbp_body(H, W, P, R, L, cs_ref, kys_ref, kxs_ref,
              x0_ref, x1_ref, x2_ref, x3_ref, out_ref):
    f = pl.program_id(0)

    acc = None
    for p, x_ref in enumerate((x0_ref, x1_ref, x2_ref, x3_ref)):
        idx = f * P + p
        ch = x_ref[:, 0]

        ky = kys_ref[idx]
        kx = kxs_ref[idx]
        s = jnp.pad(ch, ((0, 0), (0, R - H), (0, L - W)))
        s = pltpu.roll(s, ((R + _PAD) - ky) % R, 1)
        s = pltpu.roll(s, ((L + _PAD) - kx) % L, 2)
        nb = s[:, 0:H, 0:W]

        val = float(2 ** p) * jax.nn.sigmoid((nb - ch) * _INV_ALPHA)
        acc = val if acc is None else acc + val

    out_ref[:, 0] = acc


def kernel(input, kernels, projection_map):
    N, C, H, W = input.shape
    F, P = projection_map.shape

    cs = projection_map.reshape(-1).astype(jnp.int32)
    kys = kernels[..., 0].reshape(-1).astype(jnp.int32)
    kxs = kernels[..., 1].reshape(-1).astype(jnp.int32)

    rows = H + 8
    cols = W + 32

    body = functools.partial(_lbp_body, H, W, P, rows, cols)

    def _in_spec(p):
        return pl.BlockSpec(
            (N, 1, H, W),
            lambda f, cs_r, kys_r, kxs_r: (0, cs_r[f * P + p], 0, 0),
        )

    grid_spec = pltpu.PrefetchScalarGridSpec(
        num_scalar_prefetch=3,
        grid=(F,),
        in_specs=[_in_spec(p) for p in range(P)],
        out_specs=pl.BlockSpec(
            (N, 1, H, W),
            lambda f, cs_r, kys_r, kxs_r: (0, f, 0, 0),
        ),
    )

    return pl.pallas_call(
        body,
        grid_spec=grid_spec,
        out_shape=jax.ShapeDtypeStruct((N, F, H, W), jnp.float32),
        compiler_params=pltpu.CompilerParams(
            dimension_semantics=("arbitrary",),
        ),
    )(cs, kys, kxs, input, input, input, input)
